# baseline (speedup 1.0000x reference)
import functools
import jax
import jax.numpy as jnp
from jax import lax
from jax.experimental import pallas as pl
from jax.experimental.pallas import tpu as pltpu
from jax.experimental.pallas import tpu_sc as plsc

_NUM_CAT = 100
_MOST = 10000
_CDIM = 32
_B = 16384

_NPART = 4
_HC = _NUM_CAT // _NPART
_HROWS = _HC * _MOST

_NC = 2
_NS = 16
_NW = _NC * _NS
_BPW = _B // _NW
_NB = 64
_NCHUNK = _BPW // _NB
_CHUNK_IDX = _NB * _HC
_NVREG = _CHUNK_IDX // 16

_PB = 16384
_Q = _PB // 4
_QSHIFT = 12
_BSTART = tuple((q * _HROWS) // _PB for q in range(_NPART))
_NBLK = tuple(-(-((q + 1) * _HROWS) // _PB) - _BSTART[q]
              for q in range(_NPART))
_HTROWS = tuple(n * _PB for n in _NBLK)


def _make_sc_body(goff, lsub, accumulate):
    def _body(x_hbm, mask_hbm, e_hbm, *rest):
        if accumulate:
            prev_hbm = rest[0]
            rest = rest[1:]
        (out_hbm, idx0, idx1, mask0, mask1, shift_v, rows0, rows1, out_v,
         sem0, sem1) = rest
        wid = lax.axis_index("s") * _NC + lax.axis_index("c")
        base = wid * _BPW
        bufs = ((idx0, mask0, rows0, sem0), (idx1, mask1, rows1, sem1))

        def shift_body(j, _):
            pos = jnp.full((16,), j * 16, jnp.int32) + lax.iota(jnp.int32, 16)
            cat = lax.rem(pos, jnp.full((16,), _HC, jnp.int32))
            shift_v[pl.ds(j * 16, 16)] = cat * _MOST
            return 0

        lax.fori_loop(0, _NVREG, shift_body, 0)

        def stage(k, buf):
            idx_b, mask_b, rows_b, sem = buf
            b0 = base + k * _NB
            pltpu.sync_copy(x_hbm.at[pl.ds(b0 * _HC, _CHUNK_IDX)], idx_b)
            pltpu.sync_copy(mask_hbm.at[pl.ds(b0 * _HC, _CHUNK_IDX)],
                            mask_b.at[pl.ds(0, _CHUNK_IDX)])

            def add_shift(j, _):
                s = pl.ds(j * 16, 16)
                v = idx_b[s] + shift_v[s] + goff
                lo = v & (_Q - 1)
                q = (v >> _QSHIFT) & 3
                hi = v & ~(_PB - 1)
                idx_b[s] = hi + (lo << 2) + q - lsub
                return 0

            lax.fori_loop(0, _NVREG, add_shift, 0)
            pltpu.async_copy(e_hbm.at[idx_b], rows_b, sem)

        def compute(k, buf):
            idx_b, mask_b, rows_b, sem = buf
            if accumulate:
                pltpu.sync_copy(prev_hbm.at[pl.ds(base + k * _NB, _NB)],
                                out_v)
            pltpu.make_async_copy(e_hbm.at[idx_b], rows_b, sem).wait()

            def batch_body(i, _):
                ibase = i * _HC
                if accumulate:
                    a0 = out_v[i, pl.ds(0, 16)]
                    a1 = out_v[i, pl.ds(16, 16)]
                else:
                    a0 = jnp.zeros((16,), jnp.float32)
                    a1 = jnp.zeros((16,), jnp.float32)
                for blk in range(2):
                    m_vec = mask_b[pl.ds(ibase + blk * 16, 16)]
                    for lane in range(16 if blk < 1 else _HC - 16):
                        c = blk * 16 + lane
                        m = jnp.full((16,), m_vec[lane], jnp.float32)
                        a0 = a0 + m * rows_b[ibase + c, pl.ds(0, 16)]
                        a1 = a1 + m * rows_b[ibase + c, pl.ds(16, 16)]
                out_v[i, pl.ds(0, 16)] = a0
                out_v[i, pl.ds(16, 16)] = a1
                return 0

            lax.fori_loop(0, _NB, batch_body, 0)
            pltpu.sync_copy(out_v, out_hbm.at[pl.ds(base + k * _NB, _NB)])

        stage(0, bufs[0])

        def outer(t, _):
            k0 = t * 2

            @pl.when(k0 + 1 < _NCHUNK)
            def _fire1():
                stage(k0 + 1, bufs[1])

            compute(k0, bufs[0])

            @pl.when(k0 + 1 < _NCHUNK)
            def _second():
                @pl.when(k0 + 2 < _NCHUNK)
                def _fire2():
                    stage(k0 + 2, bufs[0])

                compute(k0 + 1, bufs[1])

            return 0

        lax.fori_loop(0, (_NCHUNK + 1) // 2, outer, 0)

    return _body


def _repack_body(et_ref, out_ref):
    blk = et_ref[...]
    parts = [blk[:, q * _Q:(q + 1) * _Q].T for q in range(4)]
    out_ref[...] = jnp.concatenate(parts, axis=1)


def _repack_part(et, h):
    boff = _BSTART[h]
    return pl.pallas_call(
        _repack_body,
        grid=(_NBLK[h],),
        in_specs=[pl.BlockSpec((_CDIM, _PB), lambda g, b=boff: (0, g + b))],
        out_specs=pl.BlockSpec((_Q, 128), lambda g: (g, 0)),
        out_shape=jax.ShapeDtypeStruct((_HTROWS[h] * _CDIM // 128, 128),
                                       jnp.float32),
    )(et)


def _make_sc_run(h):
    mesh = plsc.VectorSubcoreMesh(core_axis_name="c", subcore_axis_name="s")
    return functools.partial(
        pl.kernel,
        out_type=jax.ShapeDtypeStruct((_B, _CDIM), jnp.float32),
        mesh=mesh,
        compiler_params=pltpu.CompilerParams(use_tc_tiling_on_sc=False),
        scratch_types=[
            pltpu.VMEM((_CHUNK_IDX,), jnp.int32),
            pltpu.VMEM((_CHUNK_IDX,), jnp.int32),
            pltpu.VMEM((_CHUNK_IDX + 16,), jnp.float32),
            pltpu.VMEM((_CHUNK_IDX + 16,), jnp.float32),
            pltpu.VMEM((_CHUNK_IDX,), jnp.int32),
            pltpu.VMEM((_CHUNK_IDX, _CDIM), jnp.float32),
            pltpu.VMEM((_CHUNK_IDX, _CDIM), jnp.float32),
            pltpu.VMEM((_NB, _CDIM), jnp.float32),
            pltpu.SemaphoreType.DMA,
            pltpu.SemaphoreType.DMA,
        ],
    )(_make_sc_body(goff=h * _HROWS, lsub=_BSTART[h] * _PB,
                    accumulate=(h > 0)))


def kernel(x, mask, E):
    et = E.T
    out = None
    for h in range(_NPART):
        e_h = _repack_part(et, h).reshape(_HTROWS[h], _CDIM)
        x_h = x[:, h * _HC:(h + 1) * _HC].reshape(-1)
        m_h = mask[:, h * _HC:(h + 1) * _HC].reshape(-1)
        if out is None:
            out = _make_sc_run(h)(x_h, m_h, e_h)
        else:
            out = _make_sc_run(h)(x_h, m_h, e_h, out)
    return out


# speedup vs baseline: 1.0654x; 1.0654x over previous
import functools
import jax
import jax.numpy as jnp
from jax import lax
from jax.experimental import pallas as pl
from jax.experimental.pallas import tpu as pltpu
from jax.experimental.pallas import tpu_sc as plsc

_NUM_CAT = 100
_MOST = 10000
_CDIM = 32
_B = 16384

_HC = _NUM_CAT // 2
_HROWS = _HC * _MOST

_NC = 2
_NS = 16
_NW = _NC * _NS
_BPW = _B // _NW
_NB = 32
_NCHUNK = _BPW // _NB
_CHUNK_IDX = _NB * _HC
_NVREG = _CHUNK_IDX // 16

_PB = 16384
_Q = _PB // 4
_QSHIFT = 12
_BSPLIT = 30
_HBLK = (31, 32)
_HTROWS = (_HBLK[0] * _PB, _HBLK[1] * _PB)


def _make_sc_body(goff, lsub, accumulate):
    def _body(x_hbm, mask_hbm, e_hbm, *rest):
        if accumulate:
            prev_hbm = rest[0]
            rest = rest[1:]
        (out_hbm, idx0, idx1, mask0, mask1, shift_v, rows0, rows1, out_v,
         sem0, sem1) = rest
        wid = lax.axis_index("s") * _NC + lax.axis_index("c")
        base = wid * _BPW
        bufs = ((idx0, mask0, rows0, sem0), (idx1, mask1, rows1, sem1))

        def shift_body(j, _):
            pos = jnp.full((16,), j * 16, jnp.int32) + lax.iota(jnp.int32, 16)
            cat = lax.rem(pos, jnp.full((16,), _HC, jnp.int32))
            shift_v[pl.ds(j * 16, 16)] = cat * _MOST
            return 0

        lax.fori_loop(0, _NVREG, shift_body, 0)

        def stage(k, buf):
            idx_b, mask_b, rows_b, sem = buf
            b0 = base + k * _NB
            pltpu.sync_copy(x_hbm.at[pl.ds(b0 * _HC, _CHUNK_IDX)], idx_b)
            pltpu.sync_copy(mask_hbm.at[pl.ds(b0 * _HC, _CHUNK_IDX)],
                            mask_b.at[pl.ds(0, _CHUNK_IDX)])

            def add_shift(j, _):
                s = pl.ds(j * 16, 16)
                v = idx_b[s] + shift_v[s] + goff
                lo = v & (_Q - 1)
                q = (v >> _QSHIFT) & 3
                hi = v & ~(_PB - 1)
                idx_b[s] = hi + (lo << 2) + q - lsub
                return 0

            lax.fori_loop(0, _NVREG, add_shift, 0)
            pltpu.async_copy(e_hbm.at[idx_b], rows_b, sem)

        def compute(k, buf):
            idx_b, mask_b, rows_b, sem = buf
            if accumulate:
                pltpu.sync_copy(prev_hbm.at[pl.ds(base + k * _NB, _NB)],
                                out_v)
            pltpu.make_async_copy(e_hbm.at[idx_b], rows_b, sem).wait()

            def batch_body(i, _):
                ibase = i * _HC
                if accumulate:
                    a0 = out_v[i, pl.ds(0, 16)]
                    a1 = out_v[i, pl.ds(16, 16)]
                else:
                    a0 = jnp.zeros((16,), jnp.float32)
                    a1 = jnp.zeros((16,), jnp.float32)
                for blk in range(4):
                    m_vec = mask_b[pl.ds(ibase + blk * 16, 16)]
                    for lane in range(16 if blk < 3 else _HC - 48):
                        c = blk * 16 + lane
                        m = jnp.full((16,), m_vec[lane], jnp.float32)
                        a0 = a0 + m * rows_b[ibase + c, pl.ds(0, 16)]
                        a1 = a1 + m * rows_b[ibase + c, pl.ds(16, 16)]
                out_v[i, pl.ds(0, 16)] = a0
                out_v[i, pl.ds(16, 16)] = a1
                return 0

            lax.fori_loop(0, _NB, batch_body, 0)
            pltpu.sync_copy(out_v, out_hbm.at[pl.ds(base + k * _NB, _NB)])

        stage(0, bufs[0])

        def outer(t, _):
            k0 = t * 2

            @pl.when(k0 + 1 < _NCHUNK)
            def _fire1():
                stage(k0 + 1, bufs[1])

            compute(k0, bufs[0])

            @pl.when(k0 + 1 < _NCHUNK)
            def _second():
                @pl.when(k0 + 2 < _NCHUNK)
                def _fire2():
                    stage(k0 + 2, bufs[0])

                compute(k0 + 1, bufs[1])

            return 0

        lax.fori_loop(0, (_NCHUNK + 1) // 2, outer, 0)

    return _body


def _repack_body(et_ref, out_ref):
    blk = et_ref[...]
    parts = [blk[:, q * _Q:(q + 1) * _Q].T for q in range(4)]
    out_ref[...] = jnp.concatenate(parts, axis=1)


def _repack_half(et, h):
    boff = _BSPLIT * h
    return pl.pallas_call(
        _repack_body,
        grid=(_HBLK[h],),
        in_specs=[pl.BlockSpec((_CDIM, _PB), lambda g, b=boff: (0, g + b))],
        out_specs=pl.BlockSpec((_Q, 128), lambda g: (g, 0)),
        out_shape=jax.ShapeDtypeStruct((_HTROWS[h] * _CDIM // 128, 128),
                                       jnp.float32),
    )(et)


def _make_sc_run(h):
    mesh = plsc.VectorSubcoreMesh(core_axis_name="c", subcore_axis_name="s")
    return functools.partial(
        pl.kernel,
        out_type=jax.ShapeDtypeStruct((_B, _CDIM), jnp.float32),
        mesh=mesh,
        compiler_params=pltpu.CompilerParams(use_tc_tiling_on_sc=False),
        scratch_types=[
            pltpu.VMEM((_CHUNK_IDX,), jnp.int32),
            pltpu.VMEM((_CHUNK_IDX,), jnp.int32),
            pltpu.VMEM((_CHUNK_IDX + 16,), jnp.float32),
            pltpu.VMEM((_CHUNK_IDX + 16,), jnp.float32),
            pltpu.VMEM((_CHUNK_IDX,), jnp.int32),
            pltpu.VMEM((_CHUNK_IDX, _CDIM), jnp.float32),
            pltpu.VMEM((_CHUNK_IDX, _CDIM), jnp.float32),
            pltpu.VMEM((_NB, _CDIM), jnp.float32),
            pltpu.SemaphoreType.DMA,
            pltpu.SemaphoreType.DMA,
        ],
    )(_make_sc_body(goff=h * _HROWS, lsub=h * _BSPLIT * _PB,
                    accumulate=(h == 1)))


def kernel(x, mask, E):
    et = E.T
    halves = []
    for h in (0, 1):
        e_h = _repack_half(et, h).reshape(_HTROWS[h], _CDIM)
        x_h = x[:, h * _HC:(h + 1) * _HC].reshape(-1)
        m_h = mask[:, h * _HC:(h + 1) * _HC].reshape(-1)
        halves.append((x_h, m_h, e_h))
    out0 = _make_sc_run(0)(*halves[0])
    return _make_sc_run(1)(*halves[1], out0)
